# baseline (speedup 1.0000x reference)
import functools

import jax
import jax.numpy as jnp
from jax import lax
from jax.experimental import pallas as pl
from jax.experimental.pallas import tpu as pltpu
from jax.experimental.pallas import tpu_sc as plsc

N_ROWS = 320000
D = 128
S = 512
NC, NS = 2, 16
NW = NC * NS
ROWS_PER_TILE = N_ROWS // NW
C = 50
NCHUNK = ROWS_PER_TILE // C
NHALF = NCHUNK // 2
CP = 64
SEG_PER_TILE = S // NS
CW = 16
LANES = 16
NGROUP = C // LANES
TAIL = C - NGROUP * LANES


def _sc_partial_segsum(x, idxp, ones_cw, ident):
  mesh = plsc.VectorSubcoreMesh(
      core_axis_name="c", subcore_axis_name="s", num_cores=NC, num_subcores=NS)

  @functools.partial(
      pl.kernel,
      out_type=(
          jax.ShapeDtypeStruct((NC * S, D), jnp.float32),
          jax.ShapeDtypeStruct((NC * S, CW), jnp.float32),
      ),
      mesh=mesh,
      compiler_params=pltpu.CompilerParams(use_tc_tiling_on_sc=False,
                                           needs_layout_passes=False),
      scratch_types=[
          pltpu.VMEM((NCHUNK, CP), jnp.int32),
          [pltpu.VMEM((C, D), jnp.float32)] * 4,
          pltpu.VMEM((S, D), jnp.float32),
          pltpu.VMEM((CP, CW), jnp.float32),
          pltpu.VMEM((SEG_PER_TILE, CW), jnp.float32),
          pltpu.VMEM((S // 128, 128), jnp.int32),
          pltpu.VMEM_SHARED((S, D), jnp.float32),
          pltpu.VMEM_SHARED((S, CW), jnp.float32),
          [pltpu.SemaphoreType.DMA] * 4,
          [pltpu.SemaphoreType.DMA] * 2,
          pltpu.SemaphoreType.DMA,
      ],
  )
  def k(x_hbm, idxp_hbm, ones_hbm, ident_hbm, psums_hbm, pcnts_hbm,
        idx_p, xbufs, acc, ones_v, zc, ident_v, sums_sh, cnts_sh,
        gsems, csems, msem):
    cid = lax.axis_index("c")
    sid = lax.axis_index("s")
    wid = cid * NS + sid
    row0 = wid * ROWS_PER_TILE

    pltpu.sync_copy(idxp_hbm.at[wid], idx_p)
    pltpu.sync_copy(ones_hbm.at[pl.ds(0, CP)], ones_v)
    pltpu.sync_copy(ones_hbm.at[pl.ds(CP, SEG_PER_TILE)], zc)
    pltpu.sync_copy(ident_hbm, ident_v)

    zeros16 = jnp.zeros((LANES,), jnp.float32)

    def zacc(i, _):
      r = i // (D // LANES)
      acc[r, pl.ds((i % (D // LANES)) * LANES, LANES)] = zeros16
      return 0
    lax.fori_loop(0, S * (D // LANES), zacc, 0)

    pltpu.sync_copy(acc.at[pl.ds(0, SEG_PER_TILE)],
                    sums_sh.at[pl.ds(sid * SEG_PER_TILE, SEG_PER_TILE)])
    pltpu.sync_copy(zc, cnts_sh.at[pl.ds(sid * SEG_PER_TILE, SEG_PER_TILE)])
    plsc.subcore_barrier()

    def gstart(j, b):
      pltpu.async_copy(x_hbm.at[pl.ds(row0 + j * C, C)], xbufs[b], gsems[b])

    def gwait(b):
      pltpu.make_async_copy(x_hbm.at[pl.ds(0, C)], xbufs[b], gsems[b]).wait()

    lane_iota = lax.iota(jnp.int32, LANES)

    def rows_of(xb, iv, r0, u):
      segv = jnp.full((LANES,), iv[u], jnp.int32)
      for c in range(D // LANES):
        v = xb[r0 + u, pl.ds(c * LANES, LANES)]
        plsc.addupdate_scatter(acc, [segv, lane_iota + (c * LANES)], v)

    def reduce_two(jA, bA, jB, bB):
      xa, xb = xbufs[bA], xbufs[bB]

      def row_group(g, _):
        r0 = g * LANES
        iva = idx_p[jA, pl.ds(r0, LANES)]
        ivb = idx_p[jB, pl.ds(r0, LANES)]
        for u in range(LANES):
          rows_of(xa, iva, r0, u)
          rows_of(xb, ivb, r0, u)
        return 0
      lax.fori_loop(0, NGROUP, row_group, 0)
      r0 = NGROUP * LANES
      iva = idx_p[jA, pl.ds(r0, LANES)]
      ivb = idx_p[jB, pl.ds(r0, LANES)]
      for u in range(TAIL):
        rows_of(xa, iva, r0, u)
        rows_of(xb, ivb, r0, u)

    gstart(0, 0)
    gstart(NHALF, 2)

    def body(p2, _):
      for s in range(2):
        p = p2 * 2 + s
        jA = p
        jB = NHALF + p
        bA = s
        bB = 2 + s
        gwait(bA)
        gwait(bB)
        @pl.when(p < NHALF - 1)
        def _():
          gstart(jA + 1, 1 - s)
          gstart(jB + 1, 3 - s)
        dA = pltpu.async_copy(ones_v, cnts_sh.at[idx_p.at[jA]], csems[0],
                              add=True)
        dB = pltpu.async_copy(ones_v, cnts_sh.at[idx_p.at[jB]], csems[1],
                              add=True)
        reduce_two(jA, bA, jB, bB)
        dA.wait()
        dB.wait()
      return 0
    lax.fori_loop(0, NHALF // 2, body, 0)

    for q in range(S // 128):
      pltpu.async_copy(acc.at[pl.ds(q * 128, 128)],
                       sums_sh.at[ident_v.at[q]],
                       msem, add=True).wait()
    plsc.subcore_barrier()

    pltpu.sync_copy(sums_sh.at[pl.ds(sid * SEG_PER_TILE, SEG_PER_TILE)],
                    acc.at[pl.ds(0, SEG_PER_TILE)])
    pltpu.sync_copy(acc.at[pl.ds(0, SEG_PER_TILE)],
                    psums_hbm.at[pl.ds(cid * S + sid * SEG_PER_TILE,
                                       SEG_PER_TILE)])
    pltpu.sync_copy(cnts_sh.at[pl.ds(sid * SEG_PER_TILE, SEG_PER_TILE)], zc)
    pltpu.sync_copy(zc, pcnts_hbm.at[pl.ds(cid * S + sid * SEG_PER_TILE,
                                           SEG_PER_TILE)])

  return k(x, idxp, ones_cw, ident)


def _combine(psums, pcnts):
  def body(ps_ref, pc_ref, o_ref):
    sums = ps_ref[0] + ps_ref[1]
    cnts = pc_ref[0, :, 0:1] + pc_ref[1, :, 0:1]
    o_ref[...] = sums / cnts
  return pl.pallas_call(
      body,
      out_shape=jax.ShapeDtypeStruct((S, D), jnp.float32),
  )(psums.reshape(NC, S, D), pcnts.reshape(NC, S, CW))


def kernel(x, batch_index):
  idx2d = batch_index.astype(jnp.int32).reshape(NW, NCHUNK, C)
  idxp = jnp.pad(idx2d, ((0, 0), (0, 0), (0, CP - C)))
  ones_cw = jnp.concatenate([jnp.ones((C, CW), jnp.float32),
                             jnp.zeros((CP - C + SEG_PER_TILE, CW),
                                       jnp.float32)])
  ident = jnp.arange(S, dtype=jnp.int32).reshape(S // 128, 128)
  psums, pcnts = _sc_partial_segsum(x, idxp, ones_cw, ident)
  return _combine(psums, pcnts)


# speedup vs baseline: 1.7443x; 1.7443x over previous
import functools

import jax
import jax.numpy as jnp
from jax import lax
from jax.experimental import pallas as pl
from jax.experimental.pallas import tpu as pltpu
from jax.experimental.pallas import tpu_sc as plsc

N_ROWS = 320000
D = 128
S = 512
NC, NS = 2, 16
NW = NC * NS
ROWS_PER_TILE = N_ROWS // NW
C = 125
NCHUNK = ROWS_PER_TILE // C
NBUF = 4
NGRP = NCHUNK // NBUF
SEG_PER_TILE = S // NS
LANES = 16
PAD = 16


def _sc_partial_segsum(x, idx3d, ids_pad):
  mesh = plsc.VectorSubcoreMesh(
      core_axis_name="c", subcore_axis_name="s", num_cores=NC, num_subcores=NS)

  @functools.partial(
      pl.kernel,
      out_type=(
          jax.ShapeDtypeStruct((NC * S, D), jnp.float32),
          jax.ShapeDtypeStruct((NW, S), jnp.float32),
      ),
      mesh=mesh,
      compiler_params=pltpu.CompilerParams(use_tc_tiling_on_sc=False,
                                           needs_layout_passes=False),
      scratch_types=[
          pltpu.VMEM((NCHUNK, C), jnp.int32),
          pltpu.VMEM((2 * PAD + ROWS_PER_TILE,), jnp.int32),
          [pltpu.VMEM((C, D), jnp.float32)] * NBUF,
          pltpu.VMEM((S,), jnp.int32),
          pltpu.VMEM((S,), jnp.int32),
          pltpu.VMEM((S,), jnp.float32),
          pltpu.VMEM_SHARED((S, D), jnp.float32),
          [pltpu.SemaphoreType.DMA] * NBUF,
          [pltpu.SemaphoreType.DMA] * NBUF,
      ],
  )
  def k(x_hbm, idx_hbm, ids_hbm, psums_hbm, pcnts_hbm,
        idx_v, ids_v, xbufs, starts, ends, cbuf, sums_sh, gsems, ssems):
    cid = lax.axis_index("c")
    sid = lax.axis_index("s")
    wid = cid * NS + sid
    row0 = wid * ROWS_PER_TILE

    pltpu.sync_copy(idx_hbm.at[wid], idx_v)
    pltpu.sync_copy(ids_hbm.at[wid], ids_v)

    zeros16 = jnp.zeros((LANES,), jnp.float32)
    lane_iota = lax.iota(jnp.int32, LANES)

    xbuf0 = xbufs[0]
    def zrow(i, _):
      xbuf0[i // (D // LANES), pl.ds((i % (D // LANES)) * LANES, LANES)] = (
          zeros16)
      return 0
    lax.fori_loop(0, SEG_PER_TILE * (D // LANES), zrow, 0)
    pltpu.sync_copy(xbuf0.at[pl.ds(0, SEG_PER_TILE)],
                    sums_sh.at[pl.ds(sid * SEG_PER_TILE, SEG_PER_TILE)])
    plsc.subcore_barrier()

    def binit(i, _):
      starts[pl.ds(i * LANES, LANES)] = jnp.zeros((LANES,), jnp.int32)
      ends[pl.ds(i * LANES, LANES)] = jnp.full((LANES,), -1, jnp.int32)
      return 0
    lax.fori_loop(0, S // LANES, binit, 0)

    def bgroup(g, _):
      r = g * LANES
      iv = ids_v[pl.ds(PAD + r, LANES)]
      ivp = ids_v[pl.ds(PAD - 1 + r, LANES)]
      ivn = ids_v[pl.ds(PAD + 1 + r, LANES)]
      pos = r + lane_iota
      plsc.store_scatter(starts, [iv], pos, mask=iv != ivp)
      plsc.store_scatter(ends, [iv], pos, mask=iv != ivn)
      return 0
    lax.fori_loop(0, ROWS_PER_TILE // LANES, bgroup, 0)

    def ccomp(i, _):
      st = starts[pl.ds(i * LANES, LANES)]
      en = ends[pl.ds(i * LANES, LANES)]
      cnt = jnp.maximum(en - st + 1, 0)
      cbuf[pl.ds(i * LANES, LANES)] = cnt.astype(jnp.float32)
      return 0
    lax.fori_loop(0, S // LANES, ccomp, 0)
    pltpu.sync_copy(cbuf, pcnts_hbm.at[wid])

    def gstart(j, b):
      pltpu.async_copy(x_hbm.at[pl.ds(row0 + j * C, C)], xbufs[b], gsems[b])

    def gwait(b):
      pltpu.make_async_copy(x_hbm.at[pl.ds(0, C)], xbufs[b], gsems[b]).wait()

    for b in range(NBUF):
      gstart(b, b)

    def group(p, _):
      j0 = p * NBUF
      descs = []
      for b in range(NBUF):
        gwait(b)
        descs.append(pltpu.async_copy(
            xbufs[b], sums_sh.at[idx_v.at[j0 + b]], ssems[b], add=True))
      for b in range(NBUF):
        descs[b].wait()
        @pl.when(p < NGRP - 1)
        def _():
          gstart(j0 + NBUF + b, b)
      return 0
    lax.fori_loop(0, NGRP, group, 0)
    plsc.subcore_barrier()

    pltpu.sync_copy(sums_sh.at[pl.ds(sid * SEG_PER_TILE, SEG_PER_TILE)],
                    xbuf0.at[pl.ds(0, SEG_PER_TILE)])
    pltpu.sync_copy(xbuf0.at[pl.ds(0, SEG_PER_TILE)],
                    psums_hbm.at[pl.ds(cid * S + sid * SEG_PER_TILE,
                                       SEG_PER_TILE)])

  return k(x, idx3d, ids_pad)


def _combine(psums, pcnts):
  def body(ps_ref, pc_ref, o_ref):
    sums = ps_ref[0] + ps_ref[1]
    cnts = jnp.sum(pc_ref[...], axis=0)
    o_ref[...] = sums / cnts[:, None]
  return pl.pallas_call(
      body,
      out_shape=jax.ShapeDtypeStruct((S, D), jnp.float32),
  )(psums.reshape(NC, S, D), pcnts)


def kernel(x, batch_index):
  idx = batch_index.astype(jnp.int32)
  idx3d = idx.reshape(NW, NCHUNK, C)
  ids_pad = jnp.pad(idx.reshape(NW, ROWS_PER_TILE), ((0, 0), (PAD, PAD)),
                    constant_values=-1)
  psums, pcnts = _sc_partial_segsum(x, idx3d, ids_pad)
  return _combine(psums, pcnts)
